# baseline (speedup 1.0000x reference)
import functools

import jax
import jax.numpy as jnp
from jax import lax
from jax.experimental import pallas as pl
from jax.experimental.pallas import tpu as pltpu
from jax.experimental.pallas import tpu_sc as plsc

_LANES = 16
_SUB = 128
_NSUB = 8
_CHUNK = _SUB * _NSUB


@functools.partial(jax.jit, static_argnums=(2, 3))
def _sc_override_lookup(tok_flat, table, n_workers, chunks_per_worker):
    n = tok_flat.shape[0]
    v, d = table.shape
    per_w = n // n_workers
    mesh = plsc.VectorSubcoreMesh(core_axis_name="c", subcore_axis_name="s")

    @functools.partial(
        pl.kernel,
        out_type=jax.ShapeDtypeStruct((n, d), jnp.float32),
        mesh=mesh,
        compiler_params=pltpu.CompilerParams(use_tc_tiling_on_sc=False),
        scratch_types=[
            pltpu.VMEM((_CHUNK,), jnp.int32),
            pltpu.VMEM((_NSUB, _SUB), jnp.int32),
            pltpu.VMEM((_CHUNK, d), jnp.float32),
            pltpu.SemaphoreType.DMA,
        ],
    )
    def body(tok_hbm, table_hbm, out_hbm, tok_v, idx_v, rows_v, sem):
        wid = lax.axis_index("s") * 2 + lax.axis_index("c")
        base = wid * per_w

        def chunk_body(k, carry):
            off = pl.multiple_of(base + k * _CHUNK, _CHUNK)
            pltpu.sync_copy(tok_hbm.at[pl.ds(off, _CHUNK)], tok_v)

            def idx_body(g, c2):
                t = tok_v[pl.ds(pl.multiple_of(g * _LANES, _LANES), _LANES)]
                r = jnp.minimum(jnp.maximum(-t, 0), v - 1)
                s = g // (_SUB // _LANES)
                j = g % (_SUB // _LANES)
                idx_v[s, pl.ds(j * _LANES, _LANES)] = r
                return c2

            lax.fori_loop(0, _CHUNK // _LANES, idx_body, 0, unroll=4)

            copies = []
            for s in range(_NSUB):
                cp = pltpu.make_async_copy(
                    table_hbm.at[idx_v.at[s]],
                    rows_v.at[pl.ds(s * _SUB, _SUB)],
                    sem,
                )
                cp.start()
                copies.append(cp)
            for cp in copies:
                cp.wait()

            pltpu.sync_copy(rows_v, out_hbm.at[pl.ds(off, _CHUNK)])
            return carry

        lax.fori_loop(0, chunks_per_worker, chunk_body, 0)

    return body(tok_flat, table)


def kernel(tokens, wte, wte_override):
    del wte
    b, h = tokens.shape
    _, d = wte_override.shape
    n = b * h
    n_workers = 32
    per_w = n // n_workers
    assert n % n_workers == 0 and per_w % _CHUNK == 0
    tok_flat = tokens.reshape(n)
    out = _sc_override_lookup(tok_flat, wte_override, n_workers, per_w // _CHUNK)
    return out.reshape(b, h, d)


# speedup vs baseline: 29.1562x; 29.1562x over previous
import functools

import jax
import jax.numpy as jnp
from jax import lax
from jax.experimental import pallas as pl
from jax.experimental.pallas import tpu as pltpu
from jax.experimental.pallas import tpu_sc as plsc

_LANES = 16
_CHUNK = 1600
_SUPER = 8
_NW = 32
_WINFLIGHT = 4


@functools.partial(jax.jit, static_argnums=(2,))
def _sc_override_lookup(tok_flat, table_flat, d):
    n = tok_flat.shape[0]
    v = table_flat.shape[0] // d
    per_w = n // _NW
    chunks = per_w // _CHUNK
    rowbuf = _CHUNK * d
    nq = d // _LANES
    mesh = plsc.VectorSubcoreMesh(core_axis_name="c", subcore_axis_name="s")

    @functools.partial(
        pl.kernel,
        out_type=jax.ShapeDtypeStruct((n * d,), jnp.float32),
        mesh=mesh,
        compiler_params=pltpu.CompilerParams(needs_layout_passes=False),
        scratch_types=[
            pltpu.VMEM((v * d,), jnp.float32),
            pltpu.VMEM((_SUPER * _CHUNK,), jnp.int32),
            pltpu.VMEM((rowbuf,), jnp.float32),
            pltpu.SemaphoreType.DMA,
        ],
    )
    def body(tok_hbm, table_hbm, out_hbm, table_v, tok_v, rows_v, sem_out):
        wid = lax.axis_index("s") * 2 + lax.axis_index("c")
        base = wid * per_w
        pltpu.sync_copy(table_hbm, table_v)

        def fill_pattern():
            row0 = [table_v[pl.ds(q * _LANES, _LANES)] for q in range(nq)]

            def fbody(i, c2):
                o = i * d
                for q in range(nq):
                    rows_v[pl.ds(o + q * _LANES, _LANES)] = row0[q]
                return c2

            lax.fori_loop(0, _CHUNK, fbody, 0)

        fill_pattern()

        def out_copy(k):
            off = pl.multiple_of((base + k * _CHUNK) * d, 512)
            return pltpu.make_async_copy(
                rows_v, out_hbm.at[pl.ds(off, rowbuf)], sem_out
            )

        def drain(outstanding):
            def wbody(o):
                out_copy(0).wait()
                return o - 1

            return lax.while_loop(lambda o: o > 0, wbody, outstanding)

        def chunk(k, outstanding):
            @pl.when(lax.rem(k, _SUPER) == 0)
            def _():
                soff = pl.multiple_of(base + k * _CHUNK, 512)
                pltpu.sync_copy(
                    tok_hbm.at[pl.ds(soff, _SUPER * _CHUNK)], tok_v
                )

            sup_off = lax.rem(k, _SUPER) * _CHUNK

            def scan(g, m):
                toff = pl.multiple_of(sup_off + g * _LANES, _LANES)
                return jnp.minimum(m, tok_v[pl.ds(toff, _LANES)])

            m = lax.fori_loop(
                0, _CHUNK // _LANES, scan, jnp.zeros((_LANES,), jnp.int32),
                unroll=4,
            )
            any_neg = jnp.min(m) < 0

            def fast(o):
                o = lax.cond(o >= _WINFLIGHT, lambda x: drain(x - (_WINFLIGHT - 1)) + (_WINFLIGHT - 1), lambda x: x, o)
                out_copy(k).start()
                return o + 1

            def slow(o):
                o = drain(o)

                def gbody(g, c2):
                    toff = pl.multiple_of(sup_off + g * _LANES, _LANES)
                    t = tok_v[pl.ds(toff, _LANES)]
                    out0 = g * (_LANES * d)
                    for j in range(_LANES):
                        tj = t[j]
                        srcj = jnp.minimum(jnp.maximum(-tj, 0), v - 1) * d
                        oj = out0 + j * d
                        for q in range(nq):
                            rows_v[pl.ds(oj + q * _LANES, _LANES)] = table_v[
                                pl.ds(srcj + q * _LANES, _LANES)
                            ]
                    return c2

                lax.fori_loop(0, _CHUNK // _LANES, gbody, 0)
                cp = out_copy(k)
                cp.start()
                cp.wait()
                fill_pattern()
                return o

            return lax.cond(any_neg, slow, fast, outstanding)

        outstanding = lax.fori_loop(0, chunks, chunk, jnp.int32(0))
        drain(outstanding)

    return body(tok_flat, table_flat)


def kernel(tokens, wte, wte_override):
    del wte
    b, h = tokens.shape
    v, d = wte_override.shape
    n = b * h
    per_w = n // _NW
    assert n % _NW == 0 and per_w % (_CHUNK * _SUPER) == 0
    out = _sc_override_lookup(tokens.reshape(n), wte_override.reshape(v * d), d)
    return out.reshape(b, h, d)


# speedup vs baseline: 198.5351x; 6.8094x over previous
import functools

import jax
import jax.numpy as jnp
from jax import lax
from jax.experimental import pallas as pl
from jax.experimental.pallas import tpu as pltpu
from jax.experimental.pallas import tpu_sc as plsc

_LANES = 16
_UNIT = 256
_NW = 32
_WINFLIGHT = 8


@functools.partial(jax.jit, static_argnums=(2, 3))
def _sc_override_lookup(tok_flat, table_flat, h, d):
    nb = tok_flat.shape[0] // h
    v = table_flat.shape[0] // d
    units_per_w = nb // (_UNIT * _NW)
    nq = d // _LANES
    toks_per_unit = _UNIT * h
    mesh = plsc.VectorSubcoreMesh(core_axis_name="c", subcore_axis_name="s")

    @functools.partial(
        pl.kernel,
        out_type=jax.ShapeDtypeStruct((h, d, nb), jnp.float32),
        mesh=mesh,
        compiler_params=pltpu.CompilerParams(needs_layout_passes=False),
        scratch_types=[
            pltpu.VMEM((v * d,), jnp.float32),
            pltpu.VMEM((toks_per_unit,), jnp.int32),
            pltpu.VMEM((d, _UNIT), jnp.float32),
            pltpu.VMEM((d, _UNIT), jnp.float32),
            pltpu.SemaphoreType.DMA,
        ],
    )
    def body(tok_hbm, table_hbm, out_hbm, table_v, tok_v, cbuf, sbuf, sem_out):
        wid = lax.axis_index("s") * 2 + lax.axis_index("c")
        pltpu.sync_copy(table_hbm, table_v)

        for q in range(nq):
            t16 = table_v[pl.ds(q * _LANES, _LANES)]
            for l in range(_LANES):
                val = jnp.broadcast_to(t16[l], (_LANES,))
                dd = q * _LANES + l
                for j in range(_UNIT // _LANES):
                    cbuf[dd, pl.ds(j * _LANES, _LANES)] = val

        def wait_one():
            pltpu.make_async_copy(
                cbuf, out_hbm.at[0, :, pl.ds(0, _UNIT)], sem_out
            ).wait()

        def drain(outstanding):
            def wbody(o):
                wait_one()
                return o - 1

            return lax.while_loop(lambda o: o > 0, wbody, outstanding)

        lane = lax.iota(jnp.int32, _LANES)

        def unit(u, outstanding):
            b0 = u * _UNIT
            pltpu.sync_copy(
                tok_hbm.at[pl.ds(pl.multiple_of(b0 * h, 512), toks_per_unit)],
                tok_v,
            )

            def scan(g, m):
                toff = pl.multiple_of(g * _LANES, _LANES)
                return jnp.minimum(m, tok_v[pl.ds(toff, _LANES)])

            m = lax.fori_loop(
                0, toks_per_unit // _LANES, scan,
                jnp.zeros((_LANES,), jnp.int32), unroll=8,
            )
            any_neg = jnp.min(m) < 0

            def fast(o):
                def hbody(hh, o2):
                    o2 = lax.cond(
                        o2 >= _WINFLIGHT,
                        lambda x: drain(x - (_WINFLIGHT - 1)) + (_WINFLIGHT - 1),
                        lambda x: x,
                        o2,
                    )
                    pltpu.make_async_copy(
                        cbuf, out_hbm.at[hh, :, pl.ds(b0, _UNIT)], sem_out
                    ).start()
                    return o2 + 1

                return lax.fori_loop(0, h, hbody, o)

            def slow(o):
                o = drain(o)

                def hbody(hh, c2):
                    def gbody(g, c3):
                        idxs = (g * _LANES + lane) * h + hh
                        t = plsc.load_gather(tok_v, [idxs])
                        r = jnp.minimum(jnp.maximum(-t, 0), v - 1) * d
                        for l in range(_LANES):
                            rl = r[l]
                            col = jnp.broadcast_to(
                                jnp.int32(g * _LANES + l), (_LANES,)
                            )
                            for q in range(nq):
                                vals = table_v[pl.ds(rl + q * _LANES, _LANES)]
                                plsc.store_scatter(
                                    sbuf, [q * _LANES + lane, col], vals
                                )
                        return c3

                    lax.fori_loop(0, _UNIT // _LANES, gbody, 0)
                    cp = pltpu.make_async_copy(
                        sbuf, out_hbm.at[hh, :, pl.ds(b0, _UNIT)], sem_out
                    )
                    cp.start()
                    cp.wait()
                    return c2

                lax.fori_loop(0, h, hbody, 0)
                return o

            return lax.cond(any_neg, slow, fast, outstanding)

        outstanding = jnp.int32(0)
        for i in range(units_per_w):
            outstanding = unit(wid * units_per_w + i, outstanding)
        drain(outstanding)

    return body(tok_flat, table_flat)


def kernel(tokens, wte, wte_override):
    del wte
    b, h = tokens.shape
    v, d = wte_override.shape
    n = b * h
    assert b % (_UNIT * _NW) == 0 and d % _LANES == 0
    out = _sc_override_lookup(
        tokens.reshape(n), wte_override.reshape(v * d), h, d
    )
    return jnp.transpose(out, (2, 0, 1))


# speedup vs baseline: 223.2517x; 1.1245x over previous
import functools

import jax
import jax.numpy as jnp
from jax import lax
from jax.experimental import pallas as pl
from jax.experimental.pallas import tpu as pltpu
from jax.experimental.pallas import tpu_sc as plsc

_LANES = 16
_UNIT = 256
_NW = 32
_WINFLIGHT = 8


@functools.partial(jax.jit, static_argnums=(2,))
def _sc_override_lookup(tok_t, table_flat, d):
    h, nb = tok_t.shape
    v = table_flat.shape[0] // d
    units_per_w = nb // (_UNIT * _NW)
    nq = d // _LANES
    mesh = plsc.VectorSubcoreMesh(core_axis_name="c", subcore_axis_name="s")

    @functools.partial(
        pl.kernel,
        out_type=jax.ShapeDtypeStruct((h, d, nb), jnp.float32),
        mesh=mesh,
        compiler_params=pltpu.CompilerParams(needs_layout_passes=False),
        scratch_types=[
            pltpu.VMEM((v * d,), jnp.float32),
            pltpu.VMEM((h, _UNIT), jnp.int32),
            pltpu.VMEM((d, _UNIT), jnp.float32),
            pltpu.VMEM((d, _UNIT), jnp.float32),
            pltpu.SemaphoreType.DMA,
        ],
    )
    def body(tok_hbm, table_hbm, out_hbm, table_v, tok_v, cbuf, sbuf, sem_out):
        wid = lax.axis_index("s") * 2 + lax.axis_index("c")
        pltpu.sync_copy(table_hbm, table_v)

        for q in range(nq):
            t16 = table_v[pl.ds(q * _LANES, _LANES)]
            for l in range(_LANES):
                val = jnp.broadcast_to(t16[l], (_LANES,))
                dd = q * _LANES + l
                for j in range(_UNIT // _LANES):
                    cbuf[dd, pl.ds(j * _LANES, _LANES)] = val

        def wait_one():
            pltpu.make_async_copy(
                cbuf, out_hbm.at[0, :, pl.ds(0, _UNIT)], sem_out
            ).wait()

        def drain(outstanding):
            def wbody(o):
                wait_one()
                return o - 1

            return lax.while_loop(lambda o: o > 0, wbody, outstanding)

        lane = lax.iota(jnp.int32, _LANES)

        jpg = _UNIT // _LANES

        def unit(u, outstanding):
            b0 = u * _UNIT
            pltpu.sync_copy(
                tok_hbm.at[:, pl.ds(pl.multiple_of(b0, _UNIT), _UNIT)], tok_v
            )

            def scan(g, m):
                hh = lax.shift_right_logical(g, 4)
                j = lax.bitwise_and(g, jpg - 1)
                toff = pl.multiple_of(j * _LANES, _LANES)
                return jnp.minimum(m, tok_v[hh, pl.ds(toff, _LANES)])

            m = lax.fori_loop(
                0, h * jpg, scan,
                jnp.zeros((_LANES,), jnp.int32), unroll=8,
            )
            any_neg = jnp.min(m) < 0

            def fast(o):
                def hbody(hh, o2):
                    o2 = lax.cond(
                        o2 >= _WINFLIGHT,
                        lambda x: drain(x - (_WINFLIGHT - 1)) + (_WINFLIGHT - 1),
                        lambda x: x,
                        o2,
                    )
                    pltpu.make_async_copy(
                        cbuf, out_hbm.at[hh, :, pl.ds(b0, _UNIT)], sem_out
                    ).start()
                    return o2 + 1

                return lax.fori_loop(0, h, hbody, o)

            def slow(o):
                o = drain(o)

                def hbody(hh, c2):
                    def gbody(g, c3):
                        t = tok_v[hh, pl.ds(pl.multiple_of(g * _LANES, _LANES), _LANES)]
                        r = jnp.minimum(jnp.maximum(-t, 0), v - 1) * d
                        for l in range(_LANES):
                            rl = r[l]
                            col = jnp.broadcast_to(
                                jnp.int32(g * _LANES + l), (_LANES,)
                            )
                            for q in range(nq):
                                vals = table_v[pl.ds(rl + q * _LANES, _LANES)]
                                plsc.store_scatter(
                                    sbuf, [q * _LANES + lane, col], vals
                                )
                        return c3

                    lax.fori_loop(0, _UNIT // _LANES, gbody, 0)
                    cp = pltpu.make_async_copy(
                        sbuf, out_hbm.at[hh, :, pl.ds(b0, _UNIT)], sem_out
                    )
                    cp.start()
                    cp.wait()
                    return c2

                lax.fori_loop(0, h, hbody, 0)
                return o

            return lax.cond(any_neg, slow, fast, outstanding)

        outstanding = jnp.int32(0)
        for i in range(units_per_w):
            outstanding = unit(wid * units_per_w + i, outstanding)
        drain(outstanding)

    return body(tok_t, table_flat)


def kernel(tokens, wte, wte_override):
    del wte
    b, h = tokens.shape
    v, d = wte_override.shape
    assert b % (_UNIT * _NW) == 0 and d % _LANES == 0 and _UNIT == 16 * _LANES
    out = _sc_override_lookup(tokens.T, wte_override.reshape(v * d), d)
    return jnp.transpose(out, (2, 0, 1))


# speedup vs baseline: 224.1434x; 1.0040x over previous
import functools

import jax
import jax.numpy as jnp
from jax import lax
from jax.experimental import pallas as pl
from jax.experimental.pallas import tpu as pltpu
from jax.experimental.pallas import tpu_sc as plsc

_LANES = 16
_UNIT = 256
_NW = 32
_WINFLIGHT = 16


@functools.partial(jax.jit, static_argnums=(2,))
def _sc_override_lookup(tok_t, table_flat, d):
    h, nb = tok_t.shape
    v = table_flat.shape[0] // d
    units_per_w = nb // (_UNIT * _NW)
    nq = d // _LANES
    mesh = plsc.VectorSubcoreMesh(core_axis_name="c", subcore_axis_name="s")

    @functools.partial(
        pl.kernel,
        out_type=jax.ShapeDtypeStruct((h, d, nb), jnp.float32),
        mesh=mesh,
        compiler_params=pltpu.CompilerParams(needs_layout_passes=False),
        scratch_types=[
            pltpu.VMEM((v * d,), jnp.float32),
            pltpu.VMEM((h, _UNIT), jnp.int32),
            pltpu.VMEM((d, _UNIT), jnp.float32),
            pltpu.VMEM((d, _UNIT), jnp.float32),
            pltpu.SemaphoreType.DMA,
        ],
    )
    def body(tok_hbm, table_hbm, out_hbm, table_v, tok_v, cbuf, sbuf, sem_out):
        wid = lax.axis_index("s") * 2 + lax.axis_index("c")
        pltpu.sync_copy(table_hbm, table_v)

        for q in range(nq):
            t16 = table_v[pl.ds(q * _LANES, _LANES)]
            for l in range(_LANES):
                val = jnp.broadcast_to(t16[l], (_LANES,))
                dd = q * _LANES + l
                for j in range(_UNIT // _LANES):
                    cbuf[dd, pl.ds(j * _LANES, _LANES)] = val

        def wait_one():
            pltpu.make_async_copy(
                cbuf, out_hbm.at[0, :, pl.ds(0, _UNIT)], sem_out
            ).wait()

        def drain(outstanding):
            def wbody(o):
                wait_one()
                return o - 1

            return lax.while_loop(lambda o: o > 0, wbody, outstanding)

        lane = lax.iota(jnp.int32, _LANES)

        jpg = _UNIT // _LANES

        def unit(u, outstanding):
            b0 = u * _UNIT
            pltpu.sync_copy(
                tok_hbm.at[:, pl.ds(pl.multiple_of(b0, _UNIT), _UNIT)], tok_v
            )

            def scan(g, m):
                hh = lax.shift_right_logical(g, 4)
                j = lax.bitwise_and(g, jpg - 1)
                toff = pl.multiple_of(j * _LANES, _LANES)
                return jnp.minimum(m, tok_v[hh, pl.ds(toff, _LANES)])

            m = lax.fori_loop(
                0, h * jpg, scan,
                jnp.zeros((_LANES,), jnp.int32), unroll=8,
            )
            any_neg = jnp.min(m) < 0

            def fast(o):
                def hbody(hh, o2):
                    o2 = lax.cond(
                        o2 >= _WINFLIGHT,
                        lambda x: drain(x - (_WINFLIGHT - 1)) + (_WINFLIGHT - 1),
                        lambda x: x,
                        o2,
                    )
                    pltpu.make_async_copy(
                        cbuf, out_hbm.at[hh, :, pl.ds(b0, _UNIT)], sem_out
                    ).start()
                    return o2 + 1

                return lax.fori_loop(0, h, hbody, o)

            def slow(o):
                o = drain(o)

                def hbody(hh, c2):
                    def gbody(g, c3):
                        t = tok_v[hh, pl.ds(pl.multiple_of(g * _LANES, _LANES), _LANES)]
                        r = jnp.minimum(jnp.maximum(-t, 0), v - 1) * d
                        for l in range(_LANES):
                            rl = r[l]
                            col = jnp.broadcast_to(
                                jnp.int32(g * _LANES + l), (_LANES,)
                            )
                            for q in range(nq):
                                vals = table_v[pl.ds(rl + q * _LANES, _LANES)]
                                plsc.store_scatter(
                                    sbuf, [q * _LANES + lane, col], vals
                                )
                        return c3

                    lax.fori_loop(0, _UNIT // _LANES, gbody, 0)
                    cp = pltpu.make_async_copy(
                        sbuf, out_hbm.at[hh, :, pl.ds(b0, _UNIT)], sem_out
                    )
                    cp.start()
                    cp.wait()
                    return c2

                lax.fori_loop(0, h, hbody, 0)
                return o

            return lax.cond(any_neg, slow, fast, outstanding)

        outstanding = jnp.int32(0)
        for i in range(units_per_w):
            outstanding = unit(wid * units_per_w + i, outstanding)
        drain(outstanding)

    return body(tok_t, table_flat)


def kernel(tokens, wte, wte_override):
    del wte
    b, h = tokens.shape
    v, d = wte_override.shape
    assert b % (_UNIT * _NW) == 0 and d % _LANES == 0 and _UNIT == 16 * _LANES
    out = _sc_override_lookup(tokens.T, wte_override.reshape(v * d), d)
    return jnp.transpose(out, (2, 0, 1))
